# baseline (speedup 1.0000x reference)
import functools

import jax
import jax.numpy as jnp
from jax import lax
from jax.experimental import pallas as pl
from jax.experimental.pallas import tpu as pltpu
from jax.experimental.pallas import tpu_sc as plsc

LANES = 16
NC, NS = 2, 16
NW = NC * NS

EMBED = 64
GATHER_W = 128
CHUNK = 256
NBUF = 4
NG = CHUNK // GATHER_W


def _tpe_body(idx_hbm, tok_hbm, pos_hbm, out_hbm, idx_v, pos_v, rows, g_sems,
              o_sems, ld_sem, *, b_per_w, seq, nchunk):
    wid = lax.axis_index("s") * NC + lax.axis_index("c")
    b0 = wid * b_per_w

    pltpu.async_copy(idx_hbm.at[pl.ds(b0, b_per_w)], idx_v, ld_sem)
    pltpu.make_async_copy(idx_hbm.at[pl.ds(0, b_per_w)], idx_v, ld_sem).wait()
    pltpu.async_copy(pos_hbm, pos_v, ld_sem)
    pltpu.make_async_copy(pos_hbm, pos_v, ld_sem).wait()

    def gather_copies(c, k, buf):
        s0 = (k % 2) * CHUNK
        return [
            pltpu.make_async_copy(
                tok_hbm.at[idx_v.at[c // 2, pl.ds(s0 + j * GATHER_W,
                                                  GATHER_W)]],
                rows[buf].at[pl.ds(j * GATHER_W, GATHER_W)],
                g_sems[buf],
            )
            for j in range(NG)
        ]

    def out_copy(c, k, buf):
        return pltpu.make_async_copy(
            rows[buf],
            out_hbm.at[b0 + c // 2, pl.ds((k % 2) * CHUNK, CHUNK)],
            o_sems[buf],
        )

    for cp in gather_copies(0, 0, 0):
        cp.start()

    @pl.loop(0, nchunk, step=NBUF)
    def _quad(c0):
        for k in range(NBUF):
            c = c0 + k
            buf = k
            nk = (k + 1) % NBUF
            pos_off = (k % 2) * CHUNK

            @pl.when(c + 1 < nchunk)
            def _fire_next(c=c, k=k, nk=nk):
                @pl.when(c - (NBUF - 1) >= 0)
                def _drain_out(c=c, k=k, nk=nk):
                    out_copy(c - (NBUF - 1), k + 1, nk).wait()

                for cp in gather_copies(c + 1, k + 1, nk):
                    cp.start()

            for cp in gather_copies(c, k, buf):
                cp.wait()

            @pl.loop(0, CHUNK, step=8)
            def _add(r, buf=buf, pos_off=pos_off):
                for dr in range(8):
                    for cc in range(0, EMBED, LANES):
                        v = pos_v[pos_off + r + dr, pl.ds(cc, LANES)]
                        rows[buf][r + dr, pl.ds(cc, LANES)] += v

            out_copy(c, k, buf).start()

    for k in range(NBUF):
        out_copy(nchunk - NBUF + k, k, k).wait()


def kernel(x, token_table, pos_table):
    batch, seq = x.shape
    vocab, embed = token_table.shape
    b_per_w = batch // NW
    nchunk = (b_per_w * seq) // CHUNK

    idx = x.astype(jnp.int32)
    mesh = plsc.VectorSubcoreMesh(core_axis_name="c", subcore_axis_name="s")

    run = pl.kernel(
        functools.partial(_tpe_body, b_per_w=b_per_w, seq=seq,
                          nchunk=nchunk),
        out_type=jax.ShapeDtypeStruct((batch, seq, embed), jnp.float32),
        mesh=mesh,
        scratch_types=[
            pltpu.VMEM((b_per_w, seq), jnp.int32),
            pltpu.VMEM((seq, embed), jnp.float32),
            [pltpu.VMEM((CHUNK, embed), jnp.float32) for _ in range(NBUF)],
            [pltpu.SemaphoreType.DMA for _ in range(NBUF)],
            [pltpu.SemaphoreType.DMA for _ in range(NBUF)],
            pltpu.SemaphoreType.DMA,
        ],
        compiler_params=pltpu.CompilerParams(use_tc_tiling_on_sc=False),
    )
    return run(idx, token_table, pos_table)
